# baseline (speedup 1.0000x reference)
import functools

import jax
import jax.numpy as jnp
from jax import lax
from jax.experimental import pallas as pl
from jax.experimental.pallas import tpu as pltpu
from jax.experimental.pallas import tpu_sc as plsc

_N = 10000
_E = 320000
_D = 128
_K1 = 64
_K2 = 16

_NC = 2
_NS = 16
_NW = _NC * _NS
_C = 128
_NPAD = 240
_NP = _N + _NPAD
_ERP = 2560
_IR = _ERP // _NW
_G = 4
_RPS = 624
_RLAST = _NP - _NS * _RPS

_F32 = jnp.float32


def _sc_mesh():
    return plsc.VectorSubcoreMesh(core_axis_name="c", subcore_axis_name="s")


@functools.cache
def _agg_call(width):
    @functools.partial(
        pl.kernel,
        mesh=_sc_mesh(),
        out_type=jax.ShapeDtypeStruct((_NC * _NP, width), _F32),
        scratch_types=[
            pltpu.VMEM((_IR, _C), jnp.int32),
            pltpu.VMEM((_IR, _C), jnp.int32),
            pltpu.VMEM((_C, width), _F32),
            pltpu.VMEM_SHARED((_NP, width), _F32),
            pltpu.SemaphoreType.DMA,
        ],
    )
    def agg(table, gidx, sidx, zeros, out,
            gv, sv, rows, acc, gsem):
        cid = lax.axis_index("c")
        sid = lax.axis_index("s")
        wid = cid * _NS + sid
        rbase = sid * _RPS

        def stripe(src_ref, dst_ref):
            pltpu.sync_copy(src_ref.at[pl.ds(rbase, _RPS)],
                            dst_ref.at[pl.ds(rbase, _RPS)])

            @pl.when(sid == _NS - 1)
            def _():
                pltpu.sync_copy(src_ref.at[pl.ds(_NS * _RPS, _RLAST)],
                                dst_ref.at[pl.ds(_NS * _RPS, _RLAST)])

        pltpu.sync_copy(zeros, acc.at[pl.ds(rbase, _RPS)])

        @pl.when(sid == _NS - 1)
        def _():
            pltpu.sync_copy(zeros.at[pl.ds(0, _RLAST)],
                            acc.at[pl.ds(_NS * _RPS, _RLAST)])

        irow = wid * _IR
        pltpu.sync_copy(gidx.at[pl.ds(irow, _IR)], gv)
        pltpu.sync_copy(sidx.at[pl.ds(irow, _IR)], sv)
        plsc.subcore_barrier()

        @pl.loop(0, _IR)
        def _(ci):
            pltpu.async_copy(table.at[gv.at[ci]], rows, gsem).wait()
            pltpu.sync_copy(rows, acc.at[sv.at[ci]], add=True)

        plsc.subcore_barrier()
        stripe(acc, out.at[pl.ds(cid * _NP, _NP)])

    return agg


def _tc(body, out_shapes):
    return pl.pallas_call(body, out_shape=out_shapes)


def _mm_body(x_ref, w_ref, o_ref):
    o_ref[...] = jnp.dot(x_ref[...], w_ref[...], preferred_element_type=_F32)


def _dinv_body(degs_ref, h_ref, dinv_ref, t_ref):
    deg = degs_ref[: _N, 0:1] + degs_ref[_NP : _NP + _N, 0:1] + 1.0
    dinv = lax.rsqrt(deg)
    dinv_ref[...] = dinv
    t_ref[: _N] = h_ref[...] * dinv
    t_ref[_N:] = jnp.zeros((_NP - _N, _D), _F32)


def _bn_silu(y, g, b):
    mu = jnp.mean(y, axis=0, keepdims=True)
    var = jnp.mean((y - mu) ** 2, axis=0, keepdims=True)
    z = (y - mu) * lax.rsqrt(var + 1e-5) * g + b
    return z * jax.nn.sigmoid(z)


def _block1_body(acc_ref, t_ref, dinv_ref, b_ref, g_ref, be_ref, w_ref,
                 h1_ref, t1_ref):
    dinv = dinv_ref[...]
    gcn = (dinv * (acc_ref[: _N] + acc_ref[_NP : _NP + _N] + t_ref[: _N])
           + b_ref[...])
    h1 = _bn_silu(gcn, g_ref[...], be_ref[...])
    h1_ref[...] = h1
    t1_ref[: _N] = jnp.dot(h1, w_ref[...], preferred_element_type=_F32) * dinv
    t1_ref[_N:] = jnp.zeros((_NP - _N, _D), _F32)


def _block2_body(acc_ref, t_ref, dinv_ref, h1_ref, b_ref, g_ref, be_ref,
                 wp_ref, h2_ref, tp_ref):
    dinv = dinv_ref[...]
    gcn = (dinv * (acc_ref[: _N] + acc_ref[_NP : _NP + _N] + t_ref[: _N])
           + b_ref[...])
    h2 = h1_ref[...] + _bn_silu(gcn, g_ref[...], be_ref[...])
    h2_ref[...] = h2
    tp_ref[: _N] = jnp.dot(h2, wp_ref[...], preferred_element_type=_F32) * dinv
    tp_ref[_N:] = jnp.zeros((_NP - _N, _D), _F32)


def _softmax(rows):
    m = jnp.max(rows, axis=-1, keepdims=True)
    e = jnp.exp(rows - m)
    return e / jnp.sum(e, axis=-1, keepdims=True)


def _pool_body(acc_ref, tp_ref, dinv_ref, bp_ref, h2_ref, s_ref, xn_ref):
    dinv = dinv_ref[...]
    logits = (dinv * (acc_ref[: _N, : _K1] + acc_ref[_NP : _NP + _N, : _K1]
                      + tp_ref[: _N, : _K1]) + bp_ref[...])
    s = _softmax(logits)
    s_ref[: _N] = jnp.concatenate([s, jnp.zeros((_N, _D - _K1), _F32)], axis=1)
    s_ref[_N:] = jnp.zeros((_NP - _N, _D), _F32)
    xn_ref[...] = lax.dot_general(s, h2_ref[...], (((0,), (0,)), ((), ())),
                                  preferred_element_type=_F32)


def _head_body(acc_ref, s_ref, xn_ref,
               w2_ref, b2_ref, g2_ref, be2_ref,
               w3_ref, b3_ref, g3_ref, be3_ref,
               wp2_ref, bp2_ref, wl_ref, bl_ref, logp_ref):
    a_s = acc_ref[: _N, : _K1] + acc_ref[_NP : _NP + _N, : _K1]
    a_new = lax.dot_general(s_ref[: _N, : _K1], a_s, (((0,), (0,)), ((), ())),
                            preferred_element_type=_F32)
    ones = jnp.ones((_K1, 1), _F32)
    colsum = lax.dot_general(a_new, ones, (((0,), (0,)), ((), ())),
                             preferred_element_type=_F32)
    dinv2 = lax.rsqrt(colsum + 1.0)

    def conv(m, bias):
        agg = lax.dot_general(a_new, dinv2 * m, (((0,), (0,)), ((), ())),
                              preferred_element_type=_F32)
        return dinv2 * (agg + dinv2 * m) + bias

    h = xn_ref[...]
    y = conv(jnp.dot(h, w2_ref[...], preferred_element_type=_F32), b2_ref[...])
    h = _bn_silu(y, g2_ref[...], be2_ref[...])
    y = conv(jnp.dot(h, w3_ref[...], preferred_element_type=_F32), b3_ref[...])
    h = h + _bn_silu(y, g3_ref[...], be3_ref[...])
    lg = conv(jnp.dot(h, wp2_ref[...], preferred_element_type=_F32),
              bp2_ref[...])
    s2 = _softmax(lg)
    x2 = lax.dot_general(s2, h, (((0,), (0,)), ((), ())),
                         preferred_element_type=_F32)
    pooled = jnp.mean(x2, axis=0, keepdims=True)
    z = jnp.dot(pooled, wl_ref[...], preferred_element_type=_F32) + bl_ref[...]
    m = jnp.max(z, axis=-1, keepdims=True)
    lse = m + jnp.log(jnp.sum(jnp.exp(z - m), axis=-1, keepdims=True))
    logp_ref[...] = z - lse


def kernel(x, edge_index, batch, batch_ptr, params):
    p = params
    row = lambda v: v.reshape(1, -1)
    sd = lambda *s: jax.ShapeDtypeStruct(s, _F32)

    zeros_d = jnp.zeros((_RPS, _D), _F32)
    ones_t = jnp.ones((_NP, _D), _F32)
    wp1_pad = jnp.pad(p['Wp1'], ((0, 0), (0, _D - _K1)))

    npad = _ERP * _C - _E
    pad = _N + (jnp.arange(npad, dtype=jnp.int32) % _NPAD)
    src = jnp.concatenate([edge_index[0], pad]).reshape(_ERP, _C)
    dst = jnp.concatenate([edge_index[1], pad]).reshape(_ERP, _C)
    zidx = jnp.zeros((_ERP, _C), jnp.int32)

    degs = _agg_call(_D)(ones_t, zidx, dst, zeros_d)
    h0m = _tc(_mm_body, sd(_N, _D))(x, p['W0'])
    dinv, t0 = _tc(_dinv_body, [sd(_N, 1), sd(_NP, _D)])(degs, h0m)

    acc0 = _agg_call(_D)(t0, src, dst, zeros_d)
    h1, t1 = _tc(_block1_body, [sd(_N, _D), sd(_NP, _D)])(
        acc0, t0, dinv, row(p['b0']), row(p['g0']), row(p['be0']), p['W1'])

    acc1 = _agg_call(_D)(t1, src, dst, zeros_d)
    h2, tp = _tc(_block2_body, [sd(_N, _D), sd(_NP, _D)])(
        acc1, t1, dinv, h1, row(p['b1']), row(p['g1']), row(p['be1']), wp1_pad)

    accp = _agg_call(_D)(tp, src, dst, zeros_d)
    s, xn = _tc(_pool_body, [sd(_NP, _D), sd(_K1, _D)])(
        accp, tp, dinv, row(p['bp1']), h2)

    acc_as = _agg_call(_D)(s, dst, src, zeros_d)
    logp = _tc(_head_body, sd(1, 10))(
        acc_as, s, xn,
        p['W2'], row(p['b2']), row(p['g2']), row(p['be2']),
        p['W3'], row(p['b3']), row(p['g3']), row(p['be3']),
        p['Wp2'], row(p['bp2']), p['Wl'], row(p['bl']))

    return logp, jnp.zeros((), _F32)


# speedup vs baseline: 13.1961x; 13.1961x over previous
import functools

import jax
import jax.numpy as jnp
from jax import lax
from jax.experimental import pallas as pl
from jax.experimental.pallas import tpu as pltpu
from jax.experimental.pallas import tpu_sc as plsc

_N = 10000
_E = 320000
_D = 128
_K1 = 64
_K2 = 16

_NC = 2
_NS = 16
_NW = _NC * _NS
_C = 128
_NPAD = 240
_NP = _N + _NPAD
_ERP = 2560
_IR = _ERP // _NW
_G = 4
_RPS = 624
_RLAST = _NP - _NS * _RPS

_F32 = jnp.float32


def _sc_mesh():
    return plsc.VectorSubcoreMesh(core_axis_name="c", subcore_axis_name="s")


@functools.cache
def _agg_call(width):
    @functools.partial(
        pl.kernel,
        mesh=_sc_mesh(),
        out_type=jax.ShapeDtypeStruct((_NC * _NP, width), _F32),
        scratch_types=[
            pltpu.VMEM((_IR, _C), jnp.int32),
            pltpu.VMEM((_IR, _C), jnp.int32),
            pltpu.VMEM((_C, width), _F32),
            pltpu.VMEM_SHARED((_NP, width), _F32),
            pltpu.SemaphoreType.DMA,
        ],
    )
    def agg(table, gidx, sidx, zeros, out,
            gv, sv, rows, acc, gsem):
        cid = lax.axis_index("c")
        sid = lax.axis_index("s")
        wid = cid * _NS + sid
        rbase = sid * _RPS

        def stripe(src_ref, dst_ref):
            pltpu.sync_copy(src_ref.at[pl.ds(rbase, _RPS)],
                            dst_ref.at[pl.ds(rbase, _RPS)])

            @pl.when(sid == _NS - 1)
            def _():
                pltpu.sync_copy(src_ref.at[pl.ds(_NS * _RPS, _RLAST)],
                                dst_ref.at[pl.ds(_NS * _RPS, _RLAST)])

        pltpu.sync_copy(zeros, acc.at[pl.ds(rbase, _RPS)])

        @pl.when(sid == _NS - 1)
        def _():
            pltpu.sync_copy(zeros.at[pl.ds(0, _RLAST)],
                            acc.at[pl.ds(_NS * _RPS, _RLAST)])

        irow = wid * _IR
        pltpu.sync_copy(gidx.at[pl.ds(irow, _IR)], gv)
        pltpu.sync_copy(sidx.at[pl.ds(irow, _IR)], sv)
        plsc.subcore_barrier()

        @pl.loop(0, _IR)
        def _(ci):
            pltpu.async_copy(table.at[gv.at[ci]], rows, gsem).wait()
            pltpu.sync_copy(rows, acc.at[sv.at[ci]], add=True)

        plsc.subcore_barrier()
        stripe(acc, out.at[pl.ds(cid * _NP, _NP)])

    return agg


def _tc(body, out_shapes):
    return pl.pallas_call(body, out_shape=out_shapes)


def _mm_body(x_ref, w_ref, o_ref):
    o_ref[...] = jnp.dot(x_ref[...], w_ref[...], preferred_element_type=_F32)


def _dinv_body(degs_ref, h_ref, dinv_ref, t_ref):
    deg = degs_ref[: _N, 0:1] + degs_ref[_NP : _NP + _N, 0:1] + 1.0
    dinv = lax.rsqrt(deg)
    dinv_ref[...] = dinv
    t_ref[: _N] = h_ref[...] * dinv
    t_ref[_N:] = jnp.zeros((_NP - _N, _D), _F32)


def _bn_silu(y, g, b):
    mu = jnp.mean(y, axis=0, keepdims=True)
    var = jnp.mean((y - mu) ** 2, axis=0, keepdims=True)
    z = (y - mu) * lax.rsqrt(var + 1e-5) * g + b
    return z * jax.nn.sigmoid(z)


def _block1_body(acc_ref, t_ref, dinv_ref, b_ref, g_ref, be_ref, w_ref,
                 h1_ref, t1_ref):
    dinv = dinv_ref[...]
    gcn = (dinv * (acc_ref[: _N] + acc_ref[_NP : _NP + _N] + t_ref[: _N])
           + b_ref[...])
    h1 = _bn_silu(gcn, g_ref[...], be_ref[...])
    h1_ref[...] = h1
    t1_ref[: _N] = jnp.dot(h1, w_ref[...], preferred_element_type=_F32) * dinv
    t1_ref[_N:] = jnp.zeros((_NP - _N, _D), _F32)


def _block2_body(acc_ref, t_ref, dinv_ref, h1_ref, b_ref, g_ref, be_ref,
                 wp_ref, h2_ref, tp_ref):
    dinv = dinv_ref[...]
    gcn = (dinv * (acc_ref[: _N] + acc_ref[_NP : _NP + _N] + t_ref[: _N])
           + b_ref[...])
    h2 = h1_ref[...] + _bn_silu(gcn, g_ref[...], be_ref[...])
    h2_ref[...] = h2
    tp_ref[: _N] = jnp.dot(h2, wp_ref[...], preferred_element_type=_F32) * dinv
    tp_ref[_N:] = jnp.zeros((_NP - _N, _D), _F32)


def _softmax(rows):
    m = jnp.max(rows, axis=-1, keepdims=True)
    e = jnp.exp(rows - m)
    return e / jnp.sum(e, axis=-1, keepdims=True)


def _pool_body(acc_ref, tp_ref, dinv_ref, bp_ref, h2_ref, s_ref, xn_ref):
    dinv = dinv_ref[...]
    logits = (dinv * (acc_ref[: _N, : _K1] + acc_ref[_NP : _NP + _N, : _K1]
                      + tp_ref[: _N, : _K1]) + bp_ref[...])
    s = _softmax(logits)
    s_ref[: _N] = jnp.concatenate([s, jnp.zeros((_N, _D - _K1), _F32)], axis=1)
    s_ref[_N:] = jnp.zeros((_NP - _N, _D), _F32)
    xn_ref[...] = lax.dot_general(s, h2_ref[...], (((0,), (0,)), ((), ())),
                                  preferred_element_type=_F32)


def _head_body(acc_ref, s_ref, xn_ref,
               w2_ref, b2_ref, g2_ref, be2_ref,
               w3_ref, b3_ref, g3_ref, be3_ref,
               wp2_ref, bp2_ref, wl_ref, bl_ref, logp_ref):
    a_s = acc_ref[: _N, : _K1] + acc_ref[_NP : _NP + _N, : _K1]
    a_new = lax.dot_general(s_ref[: _N, : _K1], a_s, (((0,), (0,)), ((), ())),
                            preferred_element_type=_F32)
    ones = jnp.ones((_K1, 1), _F32)
    colsum = lax.dot_general(a_new, ones, (((0,), (0,)), ((), ())),
                             preferred_element_type=_F32)
    dinv2 = lax.rsqrt(colsum + 1.0)

    def conv(m, bias):
        agg = lax.dot_general(a_new, dinv2 * m, (((0,), (0,)), ((), ())),
                              preferred_element_type=_F32)
        return dinv2 * (agg + dinv2 * m) + bias

    h = xn_ref[...]
    y = conv(jnp.dot(h, w2_ref[...], preferred_element_type=_F32), b2_ref[...])
    h = _bn_silu(y, g2_ref[...], be2_ref[...])
    y = conv(jnp.dot(h, w3_ref[...], preferred_element_type=_F32), b3_ref[...])
    h = h + _bn_silu(y, g3_ref[...], be3_ref[...])
    lg = conv(jnp.dot(h, wp2_ref[...], preferred_element_type=_F32),
              bp2_ref[...])
    s2 = _softmax(lg)
    x2 = lax.dot_general(s2, h, (((0,), (0,)), ((), ())),
                         preferred_element_type=_F32)
    pooled = jnp.mean(x2, axis=0, keepdims=True)
    z = jnp.dot(pooled, wl_ref[...], preferred_element_type=_F32) + bl_ref[...]
    m = jnp.max(z, axis=-1, keepdims=True)
    lse = m + jnp.log(jnp.sum(jnp.exp(z - m), axis=-1, keepdims=True))
    logp_ref[...] = z - lse


def kernel(x, edge_index, batch, batch_ptr, params):
    p = params
    row = lambda v: v.reshape(1, -1)
    sd = lambda *s: jax.ShapeDtypeStruct(s, _F32)

    zeros_d = jnp.zeros((_RPS, _D), _F32)
    ones_t = jnp.ones((_NP, _D), _F32)
    wp1_pad = jnp.pad(p['Wp1'], ((0, 0), (0, _D - _K1)))

    npad = _ERP * _C - _E
    pad = _N + (jnp.arange(npad, dtype=jnp.int32) % _NPAD)
    src = jnp.concatenate([edge_index[0], pad]).reshape(_ERP, _C)
    dst = jnp.concatenate([edge_index[1], pad]).reshape(_ERP, _C)

    degs = _agg_call(_D)(ones_t, dst, dst, zeros_d)
    h0m = _tc(_mm_body, sd(_N, _D))(x, p['W0'])
    dinv, t0 = _tc(_dinv_body, [sd(_N, 1), sd(_NP, _D)])(degs, h0m)

    acc0 = _agg_call(_D)(t0, src, dst, zeros_d)
    h1, t1 = _tc(_block1_body, [sd(_N, _D), sd(_NP, _D)])(
        acc0, t0, dinv, row(p['b0']), row(p['g0']), row(p['be0']), p['W1'])

    acc1 = _agg_call(_D)(t1, src, dst, zeros_d)
    h2, tp = _tc(_block2_body, [sd(_N, _D), sd(_NP, _D)])(
        acc1, t1, dinv, h1, row(p['b1']), row(p['g1']), row(p['be1']), wp1_pad)

    accp = _agg_call(_D)(tp, src, dst, zeros_d)
    s, xn = _tc(_pool_body, [sd(_NP, _D), sd(_K1, _D)])(
        accp, tp, dinv, row(p['bp1']), h2)

    acc_as = _agg_call(_D)(s, dst, src, zeros_d)
    logp = _tc(_head_body, sd(1, 10))(
        acc_as, s, xn,
        p['W2'], row(p['b2']), row(p['g2']), row(p['be2']),
        p['W3'], row(p['b3']), row(p['g3']), row(p['be3']),
        p['Wp2'], row(p['bp2']), p['Wl'], row(p['bl']))

    return logp, jnp.zeros((), _F32)


# speedup vs baseline: 15.2394x; 1.1548x over previous
import functools

import jax
import jax.numpy as jnp
from jax import lax
from jax.experimental import pallas as pl
from jax.experimental.pallas import tpu as pltpu
from jax.experimental.pallas import tpu_sc as plsc

_N = 10000
_E = 320000
_D = 128
_K1 = 64
_K2 = 16

_NC = 2
_NS = 16
_NW = _NC * _NS
_C = 128
_NPAD = 240
_NP = _N + _NPAD
_ERP = 2560
_IR = _ERP // _NW
_G = 4
_RPS = 624
_RLAST = _NP - _NS * _RPS

_F32 = jnp.float32


def _sc_mesh():
    return plsc.VectorSubcoreMesh(core_axis_name="c", subcore_axis_name="s")


@functools.cache
def _agg_call(width):
    @functools.partial(
        pl.kernel,
        mesh=_sc_mesh(),
        out_type=jax.ShapeDtypeStruct((_NC * _NP, width), _F32),
        scratch_types=[
            pltpu.VMEM((_IR // 2, _C), jnp.int32),
            pltpu.VMEM((_IR // 2, _C), jnp.int32),
            pltpu.VMEM((_C, width), _F32),
            pltpu.VMEM((_C, width), _F32),
            pltpu.VMEM_SHARED((_NP, width), _F32),
            pltpu.SemaphoreType.DMA,
            pltpu.SemaphoreType.DMA,
            pltpu.SemaphoreType.DMA,
            pltpu.SemaphoreType.DMA,
        ],
    )
    def agg(table, gidx, sidx, zeros, out,
            gv, sv, rows0, rows1, acc, gsem0, gsem1, ssem0, ssem1):
        cid = lax.axis_index("c")
        sid = lax.axis_index("s")
        wid = cid * _NS + sid
        rbase = sid * _RPS

        def stripe(src_ref, dst_ref):
            pltpu.sync_copy(src_ref.at[pl.ds(rbase, _RPS)],
                            dst_ref.at[pl.ds(rbase, _RPS)])

            @pl.when(sid == _NS - 1)
            def _():
                pltpu.sync_copy(src_ref.at[pl.ds(_NS * _RPS, _RLAST)],
                                dst_ref.at[pl.ds(_NS * _RPS, _RLAST)])

        pltpu.sync_copy(zeros, acc.at[pl.ds(rbase, _RPS)])

        @pl.when(sid == _NS - 1)
        def _():
            pltpu.sync_copy(zeros.at[pl.ds(0, _RLAST)],
                            acc.at[pl.ds(_NS * _RPS, _RLAST)])

        plsc.subcore_barrier()
        irh = _IR // 2
        for h in range(2):
            irow = wid * _IR + h * irh
            pltpu.sync_copy(gidx.at[pl.ds(irow, irh)], gv)
            pltpu.sync_copy(sidx.at[pl.ds(irow, irh)], sv)

            @pl.loop(0, irh // 2)
            def _(g):
                c0 = g * 2
                h0 = pltpu.async_copy(table.at[gv.at[c0]], rows0, gsem0)
                h1 = pltpu.async_copy(table.at[gv.at[c0 + 1]], rows1, gsem1)
                h0.wait()
                w0 = pltpu.async_copy(rows0, acc.at[sv.at[c0]], ssem0,
                                      add=True)
                h1.wait()
                w1 = pltpu.async_copy(rows1, acc.at[sv.at[c0 + 1]], ssem1,
                                      add=True)
                w0.wait()
                w1.wait()

        plsc.subcore_barrier()
        stripe(acc, out.at[pl.ds(cid * _NP, _NP)])

    return agg


def _tc(body, out_shapes):
    return pl.pallas_call(body, out_shape=out_shapes)


def _mm_body(x_ref, w_ref, o_ref):
    o_ref[...] = jnp.dot(x_ref[...], w_ref[...], preferred_element_type=_F32)


def _dinv_body(degs_ref, h_ref, dinv_ref, t_ref):
    deg = degs_ref[: _N, 0:1] + degs_ref[_NP : _NP + _N, 0:1] + 1.0
    dinv = lax.rsqrt(deg)
    dinv_ref[...] = dinv
    t_ref[: _N] = h_ref[...] * dinv
    t_ref[_N:] = jnp.zeros((_NP - _N, _D), _F32)


def _bn_silu(y, g, b):
    mu = jnp.mean(y, axis=0, keepdims=True)
    var = jnp.mean((y - mu) ** 2, axis=0, keepdims=True)
    z = (y - mu) * lax.rsqrt(var + 1e-5) * g + b
    return z * jax.nn.sigmoid(z)


def _block1_body(acc_ref, t_ref, dinv_ref, b_ref, g_ref, be_ref, w_ref,
                 h1_ref, t1_ref):
    dinv = dinv_ref[...]
    gcn = (dinv * (acc_ref[: _N] + acc_ref[_NP : _NP + _N] + t_ref[: _N])
           + b_ref[...])
    h1 = _bn_silu(gcn, g_ref[...], be_ref[...])
    h1_ref[...] = h1
    t1_ref[: _N] = jnp.dot(h1, w_ref[...], preferred_element_type=_F32) * dinv
    t1_ref[_N:] = jnp.zeros((_NP - _N, _D), _F32)


def _block2_body(acc_ref, t_ref, dinv_ref, h1_ref, b_ref, g_ref, be_ref,
                 wp_ref, h2_ref, tp_ref):
    dinv = dinv_ref[...]
    gcn = (dinv * (acc_ref[: _N] + acc_ref[_NP : _NP + _N] + t_ref[: _N])
           + b_ref[...])
    h2 = h1_ref[...] + _bn_silu(gcn, g_ref[...], be_ref[...])
    h2_ref[...] = h2
    tp_ref[: _N] = jnp.dot(h2, wp_ref[...], preferred_element_type=_F32) * dinv
    tp_ref[_N:] = jnp.zeros((_NP - _N, _D), _F32)


def _softmax(rows):
    m = jnp.max(rows, axis=-1, keepdims=True)
    e = jnp.exp(rows - m)
    return e / jnp.sum(e, axis=-1, keepdims=True)


def _pool_body(acc_ref, tp_ref, dinv_ref, bp_ref, h2_ref, s_ref, xn_ref):
    dinv = dinv_ref[...]
    logits = (dinv * (acc_ref[: _N, : _K1] + acc_ref[_NP : _NP + _N, : _K1]
                      + tp_ref[: _N, : _K1]) + bp_ref[...])
    s = _softmax(logits)
    s_ref[: _N] = jnp.concatenate([s, jnp.zeros((_N, _D - _K1), _F32)], axis=1)
    s_ref[_N:] = jnp.zeros((_NP - _N, _D), _F32)
    xn_ref[...] = lax.dot_general(s, h2_ref[...], (((0,), (0,)), ((), ())),
                                  preferred_element_type=_F32)


def _head_body(acc_ref, s_ref, xn_ref,
               w2_ref, b2_ref, g2_ref, be2_ref,
               w3_ref, b3_ref, g3_ref, be3_ref,
               wp2_ref, bp2_ref, wl_ref, bl_ref, logp_ref):
    a_s = acc_ref[: _N, : _K1] + acc_ref[_NP : _NP + _N, : _K1]
    a_new = lax.dot_general(s_ref[: _N, : _K1], a_s, (((0,), (0,)), ((), ())),
                            preferred_element_type=_F32)
    ones = jnp.ones((_K1, 1), _F32)
    colsum = lax.dot_general(a_new, ones, (((0,), (0,)), ((), ())),
                             preferred_element_type=_F32)
    dinv2 = lax.rsqrt(colsum + 1.0)

    def conv(m, bias):
        agg = lax.dot_general(a_new, dinv2 * m, (((0,), (0,)), ((), ())),
                              preferred_element_type=_F32)
        return dinv2 * (agg + dinv2 * m) + bias

    h = xn_ref[...]
    y = conv(jnp.dot(h, w2_ref[...], preferred_element_type=_F32), b2_ref[...])
    h = _bn_silu(y, g2_ref[...], be2_ref[...])
    y = conv(jnp.dot(h, w3_ref[...], preferred_element_type=_F32), b3_ref[...])
    h = h + _bn_silu(y, g3_ref[...], be3_ref[...])
    lg = conv(jnp.dot(h, wp2_ref[...], preferred_element_type=_F32),
              bp2_ref[...])
    s2 = _softmax(lg)
    x2 = lax.dot_general(s2, h, (((0,), (0,)), ((), ())),
                         preferred_element_type=_F32)
    pooled = jnp.mean(x2, axis=0, keepdims=True)
    z = jnp.dot(pooled, wl_ref[...], preferred_element_type=_F32) + bl_ref[...]
    m = jnp.max(z, axis=-1, keepdims=True)
    lse = m + jnp.log(jnp.sum(jnp.exp(z - m), axis=-1, keepdims=True))
    logp_ref[...] = z - lse


def kernel(x, edge_index, batch, batch_ptr, params):
    p = params
    row = lambda v: v.reshape(1, -1)
    sd = lambda *s: jax.ShapeDtypeStruct(s, _F32)

    zeros_d = jnp.zeros((_RPS, _D), _F32)
    ones_t = jnp.ones((_NP, _D), _F32)
    wp1_pad = jnp.pad(p['Wp1'], ((0, 0), (0, _D - _K1)))

    npad = _ERP * _C - _E
    pad = _N + (jnp.arange(npad, dtype=jnp.int32) % _NPAD)
    src = jnp.concatenate([edge_index[0], pad]).reshape(_ERP, _C)
    dst = jnp.concatenate([edge_index[1], pad]).reshape(_ERP, _C)

    degs = _agg_call(_D)(ones_t, dst, dst, zeros_d)
    h0m = _tc(_mm_body, sd(_N, _D))(x, p['W0'])
    dinv, t0 = _tc(_dinv_body, [sd(_N, 1), sd(_NP, _D)])(degs, h0m)

    acc0 = _agg_call(_D)(t0, src, dst, zeros_d)
    h1, t1 = _tc(_block1_body, [sd(_N, _D), sd(_NP, _D)])(
        acc0, t0, dinv, row(p['b0']), row(p['g0']), row(p['be0']), p['W1'])

    acc1 = _agg_call(_D)(t1, src, dst, zeros_d)
    h2, tp = _tc(_block2_body, [sd(_N, _D), sd(_NP, _D)])(
        acc1, t1, dinv, h1, row(p['b1']), row(p['g1']), row(p['be1']), wp1_pad)

    accp = _agg_call(_D)(tp, src, dst, zeros_d)
    s, xn = _tc(_pool_body, [sd(_NP, _D), sd(_K1, _D)])(
        accp, tp, dinv, row(p['bp1']), h2)

    acc_as = _agg_call(_D)(s, dst, src, zeros_d)
    logp = _tc(_head_body, sd(1, 10))(
        acc_as, s, xn,
        p['W2'], row(p['b2']), row(p['g2']), row(p['be2']),
        p['W3'], row(p['b3']), row(p['g3']), row(p['be3']),
        p['Wp2'], row(p['bp2']), p['Wl'], row(p['bl']))

    return logp, jnp.zeros((), _F32)


# speedup vs baseline: 17.1534x; 1.1256x over previous
import functools

import jax
import jax.numpy as jnp
from jax import lax
from jax.experimental import pallas as pl
from jax.experimental.pallas import tpu as pltpu
from jax.experimental.pallas import tpu_sc as plsc

_N = 10000
_E = 320000
_D = 128
_K1 = 64
_K2 = 16

_NC = 2
_NS = 16
_NW = _NC * _NS
_C = 128
_NPAD = 240
_NP = _N + _NPAD
_ERP = 2560
_IR = _ERP // _NW
_G = 4
_RPS = 624
_RLAST = _NP - _NS * _RPS

_F32 = jnp.float32


def _sc_mesh():
    return plsc.VectorSubcoreMesh(core_axis_name="c", subcore_axis_name="s")


@functools.cache
def _agg_call(width, tct=True):
    @functools.partial(
        pl.kernel,
        mesh=_sc_mesh(),
        compiler_params=pltpu.CompilerParams(use_tc_tiling_on_sc=tct),
        out_type=jax.ShapeDtypeStruct((_NC * _NP, width), _F32),
        scratch_types=[
            pltpu.VMEM((_IR // 2, _C), jnp.int32),
            pltpu.VMEM((_IR // 2, _C), jnp.int32),
            pltpu.VMEM((_C, width), _F32),
            pltpu.VMEM((_C, width), _F32),
            pltpu.VMEM_SHARED((_NP, width), _F32),
            pltpu.SemaphoreType.DMA,
            pltpu.SemaphoreType.DMA,
            pltpu.SemaphoreType.DMA,
            pltpu.SemaphoreType.DMA,
        ],
    )
    def agg(table, gidx, sidx, zeros, out,
            gv, sv, rows0, rows1, acc, gsem0, gsem1, ssem0, ssem1):
        cid = lax.axis_index("c")
        sid = lax.axis_index("s")
        wid = cid * _NS + sid
        rbase = sid * _RPS

        def stripe(src_ref, dst_ref):
            pltpu.sync_copy(src_ref.at[pl.ds(rbase, _RPS)],
                            dst_ref.at[pl.ds(rbase, _RPS)])

            @pl.when(sid == _NS - 1)
            def _():
                pltpu.sync_copy(src_ref.at[pl.ds(_NS * _RPS, _RLAST)],
                                dst_ref.at[pl.ds(_NS * _RPS, _RLAST)])

        pltpu.sync_copy(zeros, acc.at[pl.ds(rbase, _RPS)])

        @pl.when(sid == _NS - 1)
        def _():
            pltpu.sync_copy(zeros.at[pl.ds(0, _RLAST)],
                            acc.at[pl.ds(_NS * _RPS, _RLAST)])

        plsc.subcore_barrier()
        irh = _IR // 2
        for h in range(2):
            irow = wid * _IR + h * irh
            pltpu.sync_copy(gidx.at[pl.ds(irow, irh)], gv)
            pltpu.sync_copy(sidx.at[pl.ds(irow, irh)], sv)

            @pl.loop(0, irh // 2)
            def _(g):
                c0 = g * 2
                h0 = pltpu.async_copy(table.at[gv.at[c0]], rows0, gsem0)
                h1 = pltpu.async_copy(table.at[gv.at[c0 + 1]], rows1, gsem1)
                h0.wait()
                w0 = pltpu.async_copy(rows0, acc.at[sv.at[c0]], ssem0,
                                      add=True)
                h1.wait()
                w1 = pltpu.async_copy(rows1, acc.at[sv.at[c0 + 1]], ssem1,
                                      add=True)
                w0.wait()
                w1.wait()

        plsc.subcore_barrier()
        stripe(acc, out.at[pl.ds(cid * _NP, _NP)])

    return agg


def _tc(body, out_shapes):
    return pl.pallas_call(body, out_shape=out_shapes)


def _mm_body(x_ref, w_ref, o_ref):
    o_ref[...] = jnp.dot(x_ref[...], w_ref[...], preferred_element_type=_F32)


def _dinv_body(degs_ref, h_ref, dinv_ref, t_ref):
    deg = degs_ref[: _N, 0:1] + degs_ref[_NP : _NP + _N, 0:1] + 1.0
    dinv = lax.rsqrt(deg)
    dinv_ref[...] = dinv
    t_ref[: _N] = h_ref[...] * dinv
    t_ref[_N:] = jnp.zeros((_NP - _N, _D), _F32)


def _bn_silu(y, g, b):
    mu = jnp.mean(y, axis=0, keepdims=True)
    var = jnp.mean((y - mu) ** 2, axis=0, keepdims=True)
    z = (y - mu) * lax.rsqrt(var + 1e-5) * g + b
    return z * jax.nn.sigmoid(z)


def _block1_body(acc_ref, t_ref, dinv_ref, b_ref, g_ref, be_ref, w_ref,
                 h1_ref, t1_ref):
    dinv = dinv_ref[...]
    gcn = (dinv * (acc_ref[: _N] + acc_ref[_NP : _NP + _N] + t_ref[: _N])
           + b_ref[...])
    h1 = _bn_silu(gcn, g_ref[...], be_ref[...])
    h1_ref[...] = h1
    t1_ref[: _N] = jnp.dot(h1, w_ref[...], preferred_element_type=_F32) * dinv
    t1_ref[_N:] = jnp.zeros((_NP - _N, _D), _F32)


def _block2_body(acc_ref, t_ref, dinv_ref, h1_ref, b_ref, g_ref, be_ref,
                 wp_ref, h2_ref, tp_ref):
    dinv = dinv_ref[...]
    gcn = (dinv * (acc_ref[: _N] + acc_ref[_NP : _NP + _N] + t_ref[: _N])
           + b_ref[...])
    h2 = h1_ref[...] + _bn_silu(gcn, g_ref[...], be_ref[...])
    h2_ref[...] = h2
    tp_ref[: _N] = jnp.dot(h2, wp_ref[...], preferred_element_type=_F32) * dinv
    tp_ref[_N:] = jnp.zeros((_NP - _N, _K1), _F32)


def _softmax(rows):
    m = jnp.max(rows, axis=-1, keepdims=True)
    e = jnp.exp(rows - m)
    return e / jnp.sum(e, axis=-1, keepdims=True)


def _pool_body(acc_ref, tp_ref, dinv_ref, bp_ref, h2_ref, s_ref, xn_ref):
    dinv = dinv_ref[...]
    logits = (dinv * (acc_ref[: _N] + acc_ref[_NP : _NP + _N]
                      + tp_ref[: _N]) + bp_ref[...])
    s = _softmax(logits)
    s_ref[: _N] = s
    s_ref[_N:] = jnp.zeros((_NP - _N, _K1), _F32)
    xn_ref[...] = lax.dot_general(s, h2_ref[...], (((0,), (0,)), ((), ())),
                                  preferred_element_type=_F32)


def _head_body(acc_ref, s_ref, xn_ref,
               w2_ref, b2_ref, g2_ref, be2_ref,
               w3_ref, b3_ref, g3_ref, be3_ref,
               wp2_ref, bp2_ref, wl_ref, bl_ref, logp_ref):
    a_s = acc_ref[: _N] + acc_ref[_NP : _NP + _N]
    a_new = lax.dot_general(s_ref[: _N], a_s, (((0,), (0,)), ((), ())),
                            preferred_element_type=_F32)
    ones = jnp.ones((_K1, 1), _F32)
    colsum = lax.dot_general(a_new, ones, (((0,), (0,)), ((), ())),
                             preferred_element_type=_F32)
    dinv2 = lax.rsqrt(colsum + 1.0)

    def conv(m, bias):
        agg = lax.dot_general(a_new, dinv2 * m, (((0,), (0,)), ((), ())),
                              preferred_element_type=_F32)
        return dinv2 * (agg + dinv2 * m) + bias

    h = xn_ref[...]
    y = conv(jnp.dot(h, w2_ref[...], preferred_element_type=_F32), b2_ref[...])
    h = _bn_silu(y, g2_ref[...], be2_ref[...])
    y = conv(jnp.dot(h, w3_ref[...], preferred_element_type=_F32), b3_ref[...])
    h = h + _bn_silu(y, g3_ref[...], be3_ref[...])
    lg = conv(jnp.dot(h, wp2_ref[...], preferred_element_type=_F32),
              bp2_ref[...])
    s2 = _softmax(lg)
    x2 = lax.dot_general(s2, h, (((0,), (0,)), ((), ())),
                         preferred_element_type=_F32)
    pooled = jnp.mean(x2, axis=0, keepdims=True)
    z = jnp.dot(pooled, wl_ref[...], preferred_element_type=_F32) + bl_ref[...]
    m = jnp.max(z, axis=-1, keepdims=True)
    lse = m + jnp.log(jnp.sum(jnp.exp(z - m), axis=-1, keepdims=True))
    logp_ref[...] = z - lse


def kernel(x, edge_index, batch, batch_ptr, params):
    p = params
    row = lambda v: v.reshape(1, -1)
    sd = lambda *s: jax.ShapeDtypeStruct(s, _F32)

    zeros_d = jnp.zeros((_RPS, _D), _F32)
    zeros_k = jnp.zeros((_RPS, _K1), _F32)
    ones_t = jnp.ones((_NP, _D), _F32)

    npad = _ERP * _C - _E
    pad = _N + (jnp.arange(npad, dtype=jnp.int32) % _NPAD)
    src = jnp.concatenate([edge_index[0], pad]).reshape(_ERP, _C)
    dst = jnp.concatenate([edge_index[1], pad]).reshape(_ERP, _C)

    degs = _agg_call(_D)(ones_t, dst, dst, zeros_d)
    h0m = _tc(_mm_body, sd(_N, _D))(x, p['W0'])
    dinv, t0 = _tc(_dinv_body, [sd(_N, 1), sd(_NP, _D)])(degs, h0m)

    acc0 = _agg_call(_D)(t0, src, dst, zeros_d)
    h1, t1 = _tc(_block1_body, [sd(_N, _D), sd(_NP, _D)])(
        acc0, t0, dinv, row(p['b0']), row(p['g0']), row(p['be0']), p['W1'])

    acc1 = _agg_call(_D)(t1, src, dst, zeros_d)
    h2, tp = _tc(_block2_body, [sd(_N, _D), sd(_NP, _K1)])(
        acc1, t1, dinv, h1, row(p['b1']), row(p['g1']), row(p['be1']),
        p['Wp1'])

    accp = _agg_call(_K1, False)(tp, src, dst, zeros_k)
    s, xn = _tc(_pool_body, [sd(_NP, _K1), sd(_K1, _D)])(
        accp, tp, dinv, row(p['bp1']), h2)

    acc_as = _agg_call(_K1, False)(s, dst, src, zeros_k)
    logp = _tc(_head_body, sd(1, 10))(
        acc_as, s, xn,
        p['W2'], row(p['b2']), row(p['g2']), row(p['be2']),
        p['W3'], row(p['b3']), row(p['g3']), row(p['be3']),
        p['Wp2'], row(p['bp2']), p['Wl'], row(p['bl']))

    return logp, jnp.zeros((), _F32)


# speedup vs baseline: 19.5831x; 1.1416x over previous
import functools

import jax
import jax.numpy as jnp
from jax import lax
from jax.experimental import pallas as pl
from jax.experimental.pallas import tpu as pltpu
from jax.experimental.pallas import tpu_sc as plsc

_N = 10000
_E = 320000
_D = 128
_K1 = 64
_K2 = 16

_NC = 2
_NS = 16
_NW = _NC * _NS
_C = 128
_NPAD = 240
_NP = _N + _NPAD
_ERP = 2560
_IR = _ERP // _NW
_G = 4
_RPS = 624
_RLAST = _NP - _NS * _RPS

_F32 = jnp.float32


def _sc_mesh():
    return plsc.VectorSubcoreMesh(core_axis_name="c", subcore_axis_name="s")


@functools.cache
def _agg_call(width, tct=True):
    @functools.partial(
        pl.kernel,
        mesh=_sc_mesh(),
        compiler_params=pltpu.CompilerParams(use_tc_tiling_on_sc=tct),
        out_type=jax.ShapeDtypeStruct((_NC * _NP, width), _F32),
        scratch_types=[
            pltpu.VMEM((_IR // 2, _C), jnp.int32),
            pltpu.VMEM((_IR // 2, _C), jnp.int32),
            pltpu.VMEM((_C, width), _F32),
            pltpu.VMEM((_C, width), _F32),
            pltpu.VMEM_SHARED((_NP, width), _F32),
            pltpu.SemaphoreType.DMA,
            pltpu.SemaphoreType.DMA,
            pltpu.SemaphoreType.DMA,
            pltpu.SemaphoreType.DMA,
        ],
    )
    def agg(table, gidx, sidx, zeros, out,
            gv, sv, rows0, rows1, acc, gsem0, gsem1, ssem0, ssem1):
        cid = lax.axis_index("c")
        sid = lax.axis_index("s")
        wid = cid * _NS + sid
        rbase = sid * _RPS

        def stripe(src_ref, dst_ref):
            pltpu.sync_copy(src_ref.at[pl.ds(rbase, _RPS)],
                            dst_ref.at[pl.ds(rbase, _RPS)])

            @pl.when(sid == _NS - 1)
            def _():
                pltpu.sync_copy(src_ref.at[pl.ds(_NS * _RPS, _RLAST)],
                                dst_ref.at[pl.ds(_NS * _RPS, _RLAST)])

        pltpu.sync_copy(zeros, acc.at[pl.ds(rbase, _RPS)])

        @pl.when(sid == _NS - 1)
        def _():
            pltpu.sync_copy(zeros.at[pl.ds(0, _RLAST)],
                            acc.at[pl.ds(_NS * _RPS, _RLAST)])

        plsc.subcore_barrier()
        irh = _IR // 2
        for h in range(2):
            irow = wid * _IR + h * irh
            pltpu.sync_copy(gidx.at[pl.ds(irow, irh)], gv)
            pltpu.sync_copy(sidx.at[pl.ds(irow, irh)], sv)

            @pl.loop(0, irh // 2)
            def _(g):
                c0 = g * 2
                h0 = pltpu.async_copy(table.at[gv.at[c0]], rows0, gsem0)
                h1 = pltpu.async_copy(table.at[gv.at[c0 + 1]], rows1, gsem1)
                h0.wait()
                w0 = pltpu.async_copy(rows0, acc.at[sv.at[c0]], ssem0,
                                      add=True)
                h1.wait()
                w1 = pltpu.async_copy(rows1, acc.at[sv.at[c0 + 1]], ssem1,
                                      add=True)
                w0.wait()
                w1.wait()

        plsc.subcore_barrier()
        stripe(acc, out.at[pl.ds(cid * _NP, _NP)])

    return agg


def _tc(body, out_shapes):
    return pl.pallas_call(body, out_shape=out_shapes)


def _mm_body(x_ref, w_ref, o_ref):
    o_ref[...] = jnp.dot(x_ref[...], w_ref[...], preferred_element_type=_F32)


def _dinv_body(degs_ref, h_ref, dinv_ref, t_ref):
    deg = degs_ref[: _N, 0:1] + degs_ref[_NP : _NP + _N, 0:1] + 1.0
    dinv = lax.rsqrt(deg)
    dinv_ref[...] = dinv
    t_ref[: _N] = h_ref[...] * dinv
    t_ref[_N:] = jnp.zeros((_NP - _N, _D), _F32)


def _bn_silu(y, g, b):
    mu = jnp.mean(y, axis=0, keepdims=True)
    var = jnp.mean((y - mu) ** 2, axis=0, keepdims=True)
    z = (y - mu) * lax.rsqrt(var + 1e-5) * g + b
    return z * jax.nn.sigmoid(z)


def _block1_body(acc_ref, t_ref, dinv_ref, b_ref, g_ref, be_ref, w_ref,
                 h1_ref, t1_ref):
    dinv = dinv_ref[...]
    gcn = (dinv * (acc_ref[: _N] + acc_ref[_NP : _NP + _N] + t_ref[: _N])
           + b_ref[...])
    h1 = _bn_silu(gcn, g_ref[...], be_ref[...])
    h1_ref[...] = h1
    t1_ref[: _N] = jnp.dot(h1, w_ref[...], preferred_element_type=_F32) * dinv
    t1_ref[_N:] = jnp.zeros((_NP - _N, _D), _F32)


def _block2_body(acc_ref, t_ref, dinv_ref, h1_ref, b_ref, g_ref, be_ref,
                 wp_ref, h2_ref, tp_ref):
    dinv = dinv_ref[...]
    gcn = (dinv * (acc_ref[: _N] + acc_ref[_NP : _NP + _N] + t_ref[: _N])
           + b_ref[...])
    h2 = h1_ref[...] + _bn_silu(gcn, g_ref[...], be_ref[...])
    h2_ref[...] = h2
    tp_ref[: _N] = jnp.dot(h2, wp_ref[...], preferred_element_type=_F32) * dinv
    tp_ref[_N:] = jnp.zeros((_NP - _N, _K1), _F32)


def _softmax(rows):
    m = jnp.max(rows, axis=-1, keepdims=True)
    e = jnp.exp(rows - m)
    return e / jnp.sum(e, axis=-1, keepdims=True)


def _pool_body(acc_ref, tp_ref, dinv_ref, bp_ref, h2_ref, s_ref, xn_ref):
    dinv = dinv_ref[...]
    logits = (dinv * (acc_ref[: _N] + acc_ref[_NP : _NP + _N]
                      + tp_ref[: _N]) + bp_ref[...])
    s = _softmax(logits)
    s_ref[: _N] = s
    s_ref[_N:] = jnp.zeros((_NP - _N, _K1), _F32)
    xn_ref[...] = lax.dot_general(s, h2_ref[...], (((0,), (0,)), ((), ())),
                                  preferred_element_type=_F32)


def _head_body(acc_ref, s_ref, xn_ref,
               w2_ref, b2_ref, g2_ref, be2_ref,
               w3_ref, b3_ref, g3_ref, be3_ref,
               wp2_ref, bp2_ref, wl_ref, bl_ref, logp_ref):
    a_s = acc_ref[: _N] + acc_ref[_NP : _NP + _N]
    a_new = lax.dot_general(s_ref[: _N], a_s, (((0,), (0,)), ((), ())),
                            preferred_element_type=_F32)
    ones = jnp.ones((_K1, 1), _F32)
    colsum = lax.dot_general(a_new, ones, (((0,), (0,)), ((), ())),
                             preferred_element_type=_F32)
    dinv2 = lax.rsqrt(colsum + 1.0)

    def conv(m, bias):
        agg = lax.dot_general(a_new, dinv2 * m, (((0,), (0,)), ((), ())),
                              preferred_element_type=_F32)
        return dinv2 * (agg + dinv2 * m) + bias

    h = xn_ref[...]
    y = conv(jnp.dot(h, w2_ref[...], preferred_element_type=_F32), b2_ref[...])
    h = _bn_silu(y, g2_ref[...], be2_ref[...])
    y = conv(jnp.dot(h, w3_ref[...], preferred_element_type=_F32), b3_ref[...])
    h = h + _bn_silu(y, g3_ref[...], be3_ref[...])
    lg = conv(jnp.dot(h, wp2_ref[...], preferred_element_type=_F32),
              bp2_ref[...])
    s2 = _softmax(lg)
    x2 = lax.dot_general(s2, h, (((0,), (0,)), ((), ())),
                         preferred_element_type=_F32)
    pooled = jnp.mean(x2, axis=0, keepdims=True)
    z = jnp.dot(pooled, wl_ref[...], preferred_element_type=_F32) + bl_ref[...]
    m = jnp.max(z, axis=-1, keepdims=True)
    lse = m + jnp.log(jnp.sum(jnp.exp(z - m), axis=-1, keepdims=True))
    logp_ref[...] = z - lse


def kernel(x, edge_index, batch, batch_ptr, params):
    p = params
    row = lambda v: v.reshape(1, -1)
    sd = lambda *s: jax.ShapeDtypeStruct(s, _F32)

    zeros_d = jnp.zeros((_RPS, _D), _F32)
    zeros_k = jnp.zeros((_RPS, _K1), _F32)
    zeros_s = jnp.zeros((_RPS, 16), _F32)
    ones_t = jnp.ones((_NP, 16), _F32)

    npad = _ERP * _C - _E
    pad = _N + (jnp.arange(npad, dtype=jnp.int32) % _NPAD)
    src = jnp.concatenate([edge_index[0], pad]).reshape(_ERP, _C)
    dst = jnp.concatenate([edge_index[1], pad]).reshape(_ERP, _C)

    degs = _agg_call(16, False)(ones_t, dst, dst, zeros_s)
    h0m = _tc(_mm_body, sd(_N, _D))(x, p['W0'])
    dinv, t0 = _tc(_dinv_body, [sd(_N, 1), sd(_NP, _D)])(degs, h0m)

    acc0 = _agg_call(_D)(t0, src, dst, zeros_d)
    h1, t1 = _tc(_block1_body, [sd(_N, _D), sd(_NP, _D)])(
        acc0, t0, dinv, row(p['b0']), row(p['g0']), row(p['be0']), p['W1'])

    acc1 = _agg_call(_D)(t1, src, dst, zeros_d)
    h2, tp = _tc(_block2_body, [sd(_N, _D), sd(_NP, _K1)])(
        acc1, t1, dinv, h1, row(p['b1']), row(p['g1']), row(p['be1']),
        p['Wp1'])

    accp = _agg_call(_K1, False)(tp, src, dst, zeros_k)
    s, xn = _tc(_pool_body, [sd(_NP, _K1), sd(_K1, _D)])(
        accp, tp, dinv, row(p['bp1']), h2)

    acc_as = _agg_call(_K1, False)(s, dst, src, zeros_k)
    logp = _tc(_head_body, sd(1, 10))(
        acc_as, s, xn,
        p['W2'], row(p['b2']), row(p['g2']), row(p['be2']),
        p['W3'], row(p['b3']), row(p['g3']), row(p['be3']),
        p['Wp2'], row(p['bp2']), p['Wl'], row(p['bl']))

    return logp, jnp.zeros((), _F32)


# speedup vs baseline: 20.5261x; 1.0482x over previous
import functools

import jax
import jax.numpy as jnp
from jax import lax
from jax.experimental import pallas as pl
from jax.experimental.pallas import tpu as pltpu
from jax.experimental.pallas import tpu_sc as plsc

_N = 10000
_E = 320000
_D = 128
_K1 = 64
_K2 = 16

_NC = 2
_NS = 16
_NW = _NC * _NS
_C = 128
_NPAD = 240
_NP = _N + _NPAD
_ERP = 2560
_IR = _ERP // _NW
_G = 4
_RPS = 624
_RLAST = _NP - _NS * _RPS

_F32 = jnp.float32


def _sc_mesh():
    return plsc.VectorSubcoreMesh(core_axis_name="c", subcore_axis_name="s")


@functools.cache
def _agg_call(width, tct=True):
    depth = 2 if width >= _D else 4

    @functools.partial(
        pl.kernel,
        mesh=_sc_mesh(),
        compiler_params=pltpu.CompilerParams(use_tc_tiling_on_sc=tct),
        out_type=jax.ShapeDtypeStruct((_NC * _NP, width), _F32),
        scratch_types=(
            [pltpu.VMEM((_IR // 2, _C), jnp.int32),
             pltpu.VMEM((_IR // 2, _C), jnp.int32)]
            + [pltpu.VMEM((_C, width), _F32)] * depth
            + [pltpu.VMEM_SHARED((_NP, width), _F32)]
            + [pltpu.SemaphoreType.DMA] * (2 * depth)
        ),
    )
    def agg(table, gidx, sidx, zeros, out, gv, sv, *rest):
        rows = rest[:depth]
        acc = rest[depth]
        gsems = rest[depth + 1 : 2 * depth + 1]
        ssems = rest[2 * depth + 1 :]
        cid = lax.axis_index("c")
        sid = lax.axis_index("s")
        wid = cid * _NS + sid
        rbase = sid * _RPS

        def stripe(src_ref, dst_ref):
            pltpu.sync_copy(src_ref.at[pl.ds(rbase, _RPS)],
                            dst_ref.at[pl.ds(rbase, _RPS)])

            @pl.when(sid == _NS - 1)
            def _():
                pltpu.sync_copy(src_ref.at[pl.ds(_NS * _RPS, _RLAST)],
                                dst_ref.at[pl.ds(_NS * _RPS, _RLAST)])

        pltpu.sync_copy(zeros, acc.at[pl.ds(rbase, _RPS)])

        @pl.when(sid == _NS - 1)
        def _():
            pltpu.sync_copy(zeros.at[pl.ds(0, _RLAST)],
                            acc.at[pl.ds(_NS * _RPS, _RLAST)])

        plsc.subcore_barrier()
        irh = _IR // 2
        for h in range(2):
            irow = wid * _IR + h * irh
            pltpu.sync_copy(gidx.at[pl.ds(irow, irh)], gv)
            pltpu.sync_copy(sidx.at[pl.ds(irow, irh)], sv)

            @pl.loop(0, irh // depth)
            def _(g):
                c0 = g * depth
                hs = [pltpu.async_copy(table.at[gv.at[c0 + k]], rows[k],
                                       gsems[k]) for k in range(depth)]
                ws = []
                for k in range(depth):
                    hs[k].wait()
                    ws.append(pltpu.async_copy(rows[k], acc.at[sv.at[c0 + k]],
                                               ssems[k], add=True))
                for w in ws:
                    w.wait()

        plsc.subcore_barrier()
        stripe(acc, out.at[pl.ds(cid * _NP, _NP)])

    return agg


def _tc(body, out_shapes):
    return pl.pallas_call(body, out_shape=out_shapes)


def _mm_body(x_ref, w_ref, o_ref):
    o_ref[...] = jnp.dot(x_ref[...], w_ref[...], preferred_element_type=_F32)


def _dinv_body(degs_ref, h_ref, dinv_ref, t_ref):
    deg = degs_ref[: _N, 0:1] + degs_ref[_NP : _NP + _N, 0:1] + 1.0
    dinv = lax.rsqrt(deg)
    dinv_ref[...] = dinv
    t_ref[: _N] = h_ref[...] * dinv
    t_ref[_N:] = jnp.zeros((_NP - _N, _D), _F32)


def _bn_silu(y, g, b):
    mu = jnp.mean(y, axis=0, keepdims=True)
    var = jnp.mean((y - mu) ** 2, axis=0, keepdims=True)
    z = (y - mu) * lax.rsqrt(var + 1e-5) * g + b
    return z * jax.nn.sigmoid(z)


def _block1_body(acc_ref, t_ref, dinv_ref, b_ref, g_ref, be_ref, w_ref,
                 h1_ref, t1_ref):
    dinv = dinv_ref[...]
    gcn = (dinv * (acc_ref[: _N] + acc_ref[_NP : _NP + _N] + t_ref[: _N])
           + b_ref[...])
    h1 = _bn_silu(gcn, g_ref[...], be_ref[...])
    h1_ref[...] = h1
    t1_ref[: _N] = jnp.dot(h1, w_ref[...], preferred_element_type=_F32) * dinv
    t1_ref[_N:] = jnp.zeros((_NP - _N, _D), _F32)


def _block2_body(acc_ref, t_ref, dinv_ref, h1_ref, b_ref, g_ref, be_ref,
                 wp_ref, h2_ref, tp_ref):
    dinv = dinv_ref[...]
    gcn = (dinv * (acc_ref[: _N] + acc_ref[_NP : _NP + _N] + t_ref[: _N])
           + b_ref[...])
    h2 = h1_ref[...] + _bn_silu(gcn, g_ref[...], be_ref[...])
    h2_ref[...] = h2
    tp_ref[: _N] = jnp.dot(h2, wp_ref[...], preferred_element_type=_F32) * dinv
    tp_ref[_N:] = jnp.zeros((_NP - _N, _K1), _F32)


def _softmax(rows):
    m = jnp.max(rows, axis=-1, keepdims=True)
    e = jnp.exp(rows - m)
    return e / jnp.sum(e, axis=-1, keepdims=True)


def _pool_body(acc_ref, tp_ref, dinv_ref, bp_ref, h2_ref, s_ref, xn_ref):
    dinv = dinv_ref[...]
    logits = (dinv * (acc_ref[: _N] + acc_ref[_NP : _NP + _N]
                      + tp_ref[: _N]) + bp_ref[...])
    s = _softmax(logits)
    s_ref[: _N] = s
    s_ref[_N:] = jnp.zeros((_NP - _N, _K1), _F32)
    xn_ref[...] = lax.dot_general(s, h2_ref[...], (((0,), (0,)), ((), ())),
                                  preferred_element_type=_F32)


def _head_body(acc_ref, s_ref, xn_ref,
               w2_ref, b2_ref, g2_ref, be2_ref,
               w3_ref, b3_ref, g3_ref, be3_ref,
               wp2_ref, bp2_ref, wl_ref, bl_ref, logp_ref):
    a_s = acc_ref[: _N] + acc_ref[_NP : _NP + _N]
    a_new = lax.dot_general(s_ref[: _N], a_s, (((0,), (0,)), ((), ())),
                            preferred_element_type=_F32)
    ones = jnp.ones((_K1, 1), _F32)
    colsum = lax.dot_general(a_new, ones, (((0,), (0,)), ((), ())),
                             preferred_element_type=_F32)
    dinv2 = lax.rsqrt(colsum + 1.0)

    def conv(m, bias):
        agg = lax.dot_general(a_new, dinv2 * m, (((0,), (0,)), ((), ())),
                              preferred_element_type=_F32)
        return dinv2 * (agg + dinv2 * m) + bias

    h = xn_ref[...]
    y = conv(jnp.dot(h, w2_ref[...], preferred_element_type=_F32), b2_ref[...])
    h = _bn_silu(y, g2_ref[...], be2_ref[...])
    y = conv(jnp.dot(h, w3_ref[...], preferred_element_type=_F32), b3_ref[...])
    h = h + _bn_silu(y, g3_ref[...], be3_ref[...])
    lg = conv(jnp.dot(h, wp2_ref[...], preferred_element_type=_F32),
              bp2_ref[...])
    s2 = _softmax(lg)
    x2 = lax.dot_general(s2, h, (((0,), (0,)), ((), ())),
                         preferred_element_type=_F32)
    pooled = jnp.mean(x2, axis=0, keepdims=True)
    z = jnp.dot(pooled, wl_ref[...], preferred_element_type=_F32) + bl_ref[...]
    m = jnp.max(z, axis=-1, keepdims=True)
    lse = m + jnp.log(jnp.sum(jnp.exp(z - m), axis=-1, keepdims=True))
    logp_ref[...] = z - lse


def kernel(x, edge_index, batch, batch_ptr, params):
    p = params
    row = lambda v: v.reshape(1, -1)
    sd = lambda *s: jax.ShapeDtypeStruct(s, _F32)

    zeros_d = jnp.zeros((_RPS, _D), _F32)
    zeros_k = jnp.zeros((_RPS, _K1), _F32)
    zeros_s = jnp.zeros((_RPS, 16), _F32)
    ones_t = jnp.ones((_NP, 16), _F32)

    npad = _ERP * _C - _E
    pad = _N + (jnp.arange(npad, dtype=jnp.int32) % _NPAD)
    src = jnp.concatenate([edge_index[0], pad]).reshape(_ERP, _C)
    dst = jnp.concatenate([edge_index[1], pad]).reshape(_ERP, _C)

    degs = _agg_call(16, False)(ones_t, dst, dst, zeros_s)
    h0m = _tc(_mm_body, sd(_N, _D))(x, p['W0'])
    dinv, t0 = _tc(_dinv_body, [sd(_N, 1), sd(_NP, _D)])(degs, h0m)

    acc0 = _agg_call(_D)(t0, src, dst, zeros_d)
    h1, t1 = _tc(_block1_body, [sd(_N, _D), sd(_NP, _D)])(
        acc0, t0, dinv, row(p['b0']), row(p['g0']), row(p['be0']), p['W1'])

    acc1 = _agg_call(_D)(t1, src, dst, zeros_d)
    h2, tp = _tc(_block2_body, [sd(_N, _D), sd(_NP, _K1)])(
        acc1, t1, dinv, h1, row(p['b1']), row(p['g1']), row(p['be1']),
        p['Wp1'])

    accp = _agg_call(_K1, False)(tp, src, dst, zeros_k)
    s, xn = _tc(_pool_body, [sd(_NP, _K1), sd(_K1, _D)])(
        accp, tp, dinv, row(p['bp1']), h2)

    acc_as = _agg_call(_K1, False)(s, dst, src, zeros_k)
    logp = _tc(_head_body, sd(1, 10))(
        acc_as, s, xn,
        p['W2'], row(p['b2']), row(p['g2']), row(p['be2']),
        p['W3'], row(p['b3']), row(p['g3']), row(p['be3']),
        p['Wp2'], row(p['bp2']), p['Wl'], row(p['bl']))

    return logp, jnp.zeros((), _F32)


# speedup vs baseline: 20.6552x; 1.0063x over previous
import functools

import jax
import jax.numpy as jnp
from jax import lax
from jax.experimental import pallas as pl
from jax.experimental.pallas import tpu as pltpu
from jax.experimental.pallas import tpu_sc as plsc

_N = 10000
_E = 320000
_D = 128
_K1 = 64
_K2 = 16

_NC = 2
_NS = 16
_NW = _NC * _NS
_C = 128
_NPAD = 240
_NP = _N + _NPAD
_ERP = 2560
_IR = _ERP // _NW
_G = 4
_RPS = 624
_RLAST = _NP - _NS * _RPS

_F32 = jnp.float32


def _sc_mesh():
    return plsc.VectorSubcoreMesh(core_axis_name="c", subcore_axis_name="s")


@functools.cache
def _agg_call(width, tct=True):
    depth = 2 if width >= _D else 4

    @functools.partial(
        pl.kernel,
        mesh=_sc_mesh(),
        compiler_params=pltpu.CompilerParams(use_tc_tiling_on_sc=tct),
        out_type=jax.ShapeDtypeStruct((_NC * _NP, width), _F32),
        scratch_types=(
            [pltpu.VMEM((_IR // 2, _C), jnp.int32),
             pltpu.VMEM((_IR // 2, _C), jnp.int32)]
            + [pltpu.VMEM((_C, width), _F32)] * depth
            + [pltpu.VMEM_SHARED((_NP, width), _F32)]
            + [pltpu.SemaphoreType.DMA] * (2 * depth)
        ),
    )
    def agg(table, gidx, sidx, zeros, out, gv, sv, *rest):
        rows = rest[:depth]
        acc = rest[depth]
        gsems = rest[depth + 1 : 2 * depth + 1]
        ssems = rest[2 * depth + 1 :]
        cid = lax.axis_index("c")
        sid = lax.axis_index("s")
        wid = cid * _NS + sid
        rbase = sid * _RPS

        def stripe(src_ref, dst_ref):
            pltpu.sync_copy(src_ref.at[pl.ds(rbase, _RPS)],
                            dst_ref.at[pl.ds(rbase, _RPS)])

            @pl.when(sid == _NS - 1)
            def _():
                pltpu.sync_copy(src_ref.at[pl.ds(_NS * _RPS, _RLAST)],
                                dst_ref.at[pl.ds(_NS * _RPS, _RLAST)])

        pltpu.sync_copy(zeros, acc.at[pl.ds(rbase, _RPS)])

        @pl.when(sid == _NS - 1)
        def _():
            pltpu.sync_copy(zeros.at[pl.ds(0, _RLAST)],
                            acc.at[pl.ds(_NS * _RPS, _RLAST)])

        plsc.subcore_barrier()
        irh = _IR // 2
        for h in range(2):
            irow = wid * _IR + h * irh
            pltpu.sync_copy(gidx.at[pl.ds(irow, irh)], gv)
            pltpu.sync_copy(sidx.at[pl.ds(irow, irh)], sv)

            @pl.loop(0, irh // depth)
            def _(g):
                c0 = g * depth
                hs = [pltpu.async_copy(table.at[gv.at[c0 + k]], rows[k],
                                       gsems[k]) for k in range(depth)]
                ws = []
                for k in range(depth):
                    hs[k].wait()
                    ws.append(pltpu.async_copy(rows[k], acc.at[sv.at[c0 + k]],
                                               ssems[k], add=True))
                for w in ws:
                    w.wait()

        plsc.subcore_barrier()
        stripe(acc, out.at[pl.ds(cid * _NP, _NP)])

    return agg


def _tc(body, out_shapes):
    return pl.pallas_call(body, out_shape=out_shapes)


def _dinv_body(degs_ref, x_ref, w_ref, dinv_ref, t_ref):
    deg = degs_ref[: _N, 0:1] + degs_ref[_NP : _NP + _N, 0:1] + 1.0
    dinv = lax.rsqrt(deg)
    dinv_ref[...] = dinv
    h = jnp.dot(x_ref[...], w_ref[...], preferred_element_type=_F32)
    t_ref[: _N] = h * dinv
    t_ref[_N:] = jnp.zeros((_NP - _N, _D), _F32)


def _bn_silu(y, g, b):
    mu = jnp.mean(y, axis=0, keepdims=True)
    var = jnp.mean((y - mu) ** 2, axis=0, keepdims=True)
    z = (y - mu) * lax.rsqrt(var + 1e-5) * g + b
    return z * jax.nn.sigmoid(z)


def _block1_body(acc_ref, t_ref, dinv_ref, b_ref, g_ref, be_ref, w_ref,
                 h1_ref, t1_ref):
    dinv = dinv_ref[...]
    gcn = (dinv * (acc_ref[: _N] + acc_ref[_NP : _NP + _N] + t_ref[: _N])
           + b_ref[...])
    h1 = _bn_silu(gcn, g_ref[...], be_ref[...])
    h1_ref[...] = h1
    t1_ref[: _N] = jnp.dot(h1, w_ref[...], preferred_element_type=_F32) * dinv
    t1_ref[_N:] = jnp.zeros((_NP - _N, _D), _F32)


def _block2_body(acc_ref, t_ref, dinv_ref, h1_ref, b_ref, g_ref, be_ref,
                 wp_ref, h2_ref, tp_ref):
    dinv = dinv_ref[...]
    gcn = (dinv * (acc_ref[: _N] + acc_ref[_NP : _NP + _N] + t_ref[: _N])
           + b_ref[...])
    h2 = h1_ref[...] + _bn_silu(gcn, g_ref[...], be_ref[...])
    h2_ref[...] = h2
    tp_ref[: _N] = jnp.dot(h2, wp_ref[...], preferred_element_type=_F32) * dinv
    tp_ref[_N:] = jnp.zeros((_NP - _N, _K1), _F32)


def _softmax(rows):
    m = jnp.max(rows, axis=-1, keepdims=True)
    e = jnp.exp(rows - m)
    return e / jnp.sum(e, axis=-1, keepdims=True)


def _pool_body(acc_ref, tp_ref, dinv_ref, bp_ref, h2_ref, s_ref, xn_ref):
    dinv = dinv_ref[...]
    logits = (dinv * (acc_ref[: _N] + acc_ref[_NP : _NP + _N]
                      + tp_ref[: _N]) + bp_ref[...])
    s = _softmax(logits)
    s_ref[: _N] = s
    s_ref[_N:] = jnp.zeros((_NP - _N, _K1), _F32)
    xn_ref[...] = lax.dot_general(s, h2_ref[...], (((0,), (0,)), ((), ())),
                                  preferred_element_type=_F32)


def _head_body(acc_ref, s_ref, xn_ref,
               w2_ref, b2_ref, g2_ref, be2_ref,
               w3_ref, b3_ref, g3_ref, be3_ref,
               wp2_ref, bp2_ref, wl_ref, bl_ref, logp_ref):
    a_s = acc_ref[: _N] + acc_ref[_NP : _NP + _N]
    a_new = lax.dot_general(s_ref[: _N], a_s, (((0,), (0,)), ((), ())),
                            preferred_element_type=_F32)
    ones = jnp.ones((_K1, 1), _F32)
    colsum = lax.dot_general(a_new, ones, (((0,), (0,)), ((), ())),
                             preferred_element_type=_F32)
    dinv2 = lax.rsqrt(colsum + 1.0)

    def conv(m, bias):
        agg = lax.dot_general(a_new, dinv2 * m, (((0,), (0,)), ((), ())),
                              preferred_element_type=_F32)
        return dinv2 * (agg + dinv2 * m) + bias

    h = xn_ref[...]
    y = conv(jnp.dot(h, w2_ref[...], preferred_element_type=_F32), b2_ref[...])
    h = _bn_silu(y, g2_ref[...], be2_ref[...])
    y = conv(jnp.dot(h, w3_ref[...], preferred_element_type=_F32), b3_ref[...])
    h = h + _bn_silu(y, g3_ref[...], be3_ref[...])
    lg = conv(jnp.dot(h, wp2_ref[...], preferred_element_type=_F32),
              bp2_ref[...])
    s2 = _softmax(lg)
    x2 = lax.dot_general(s2, h, (((0,), (0,)), ((), ())),
                         preferred_element_type=_F32)
    pooled = jnp.mean(x2, axis=0, keepdims=True)
    z = jnp.dot(pooled, wl_ref[...], preferred_element_type=_F32) + bl_ref[...]
    m = jnp.max(z, axis=-1, keepdims=True)
    lse = m + jnp.log(jnp.sum(jnp.exp(z - m), axis=-1, keepdims=True))
    logp_ref[...] = z - lse


def kernel(x, edge_index, batch, batch_ptr, params):
    p = params
    row = lambda v: v.reshape(1, -1)
    sd = lambda *s: jax.ShapeDtypeStruct(s, _F32)

    zeros_d = jnp.zeros((_RPS, _D), _F32)
    zeros_k = jnp.zeros((_RPS, _K1), _F32)
    zeros_s = jnp.zeros((_RPS, 16), _F32)
    ones_t = jnp.ones((_NP, 16), _F32)

    npad = _ERP * _C - _E
    pad = _N + (jnp.arange(npad, dtype=jnp.int32) % _NPAD)
    src = jnp.concatenate([edge_index[0], pad]).reshape(_ERP, _C)
    dst = jnp.concatenate([edge_index[1], pad]).reshape(_ERP, _C)

    degs = _agg_call(16, False)(ones_t, dst, dst, zeros_s)
    dinv, t0 = _tc(_dinv_body, [sd(_N, 1), sd(_NP, _D)])(degs, x, p['W0'])

    acc0 = _agg_call(_D)(t0, src, dst, zeros_d)
    h1, t1 = _tc(_block1_body, [sd(_N, _D), sd(_NP, _D)])(
        acc0, t0, dinv, row(p['b0']), row(p['g0']), row(p['be0']), p['W1'])

    acc1 = _agg_call(_D)(t1, src, dst, zeros_d)
    h2, tp = _tc(_block2_body, [sd(_N, _D), sd(_NP, _K1)])(
        acc1, t1, dinv, h1, row(p['b1']), row(p['g1']), row(p['be1']),
        p['Wp1'])

    accp = _agg_call(_K1, False)(tp, src, dst, zeros_k)
    s, xn = _tc(_pool_body, [sd(_NP, _K1), sd(_K1, _D)])(
        accp, tp, dinv, row(p['bp1']), h2)

    acc_as = _agg_call(_K1, False)(s, dst, src, zeros_k)
    logp = _tc(_head_body, sd(1, 10))(
        acc_as, s, xn,
        p['W2'], row(p['b2']), row(p['g2']), row(p['be2']),
        p['W3'], row(p['b3']), row(p['g3']), row(p['be3']),
        p['Wp2'], row(p['bp2']), p['Wl'], row(p['bl']))

    return logp, jnp.zeros((), _F32)


# speedup vs baseline: 20.6922x; 1.0018x over previous
import functools

import jax
import jax.numpy as jnp
from jax import lax
from jax.experimental import pallas as pl
from jax.experimental.pallas import tpu as pltpu
from jax.experimental.pallas import tpu_sc as plsc

_N = 10000
_E = 320000
_D = 128
_K1 = 64
_K2 = 16

_NC = 2
_NS = 16
_NW = _NC * _NS
_C = 128
_NPAD = 240
_NP = _N + _NPAD
_ERP = 2560
_IR = _ERP // _NW
_RPS = 624
_RLAST = _NP - _NS * _RPS

_F32 = jnp.float32


def _sc_mesh():
    return plsc.VectorSubcoreMesh(core_axis_name="c", subcore_axis_name="s")


@functools.cache
def _agg_call(width, tct=True):
    depth = 2 if width >= _D else 4

    @functools.partial(
        pl.kernel,
        mesh=_sc_mesh(),
        compiler_params=pltpu.CompilerParams(use_tc_tiling_on_sc=tct),
        out_type=jax.ShapeDtypeStruct((_NC * _NP, width), _F32),
        scratch_types=(
            [pltpu.VMEM((_IR // 2, _C), jnp.int32),
             pltpu.VMEM((_IR // 2, _C), jnp.int32)]
            + [pltpu.VMEM((_C, width), _F32)] * depth
            + [pltpu.VMEM_SHARED((_NP, width), _F32)]
            + [pltpu.SemaphoreType.DMA] * (2 * depth)
        ),
    )
    def agg(table, gidx, sidx, zeros, out, gv, sv, *rest):
        rows = rest[:depth]
        acc = rest[depth]
        gsems = rest[depth + 1 : 2 * depth + 1]
        ssems = rest[2 * depth + 1 :]
        cid = lax.axis_index("c")
        sid = lax.axis_index("s")
        wid = cid * _NS + sid
        rbase = sid * _RPS

        def stripe(src_ref, dst_ref):
            pltpu.sync_copy(src_ref.at[pl.ds(rbase, _RPS)],
                            dst_ref.at[pl.ds(rbase, _RPS)])

            @pl.when(sid == _NS - 1)
            def _():
                pltpu.sync_copy(src_ref.at[pl.ds(_NS * _RPS, _RLAST)],
                                dst_ref.at[pl.ds(_NS * _RPS, _RLAST)])

        pltpu.sync_copy(zeros, acc.at[pl.ds(rbase, _RPS)])

        @pl.when(sid == _NS - 1)
        def _():
            pltpu.sync_copy(zeros.at[pl.ds(0, _RLAST)],
                            acc.at[pl.ds(_NS * _RPS, _RLAST)])

        plsc.subcore_barrier()
        irh = _IR // 2
        for h in range(2):
            irow = wid * _IR + h * irh
            pltpu.sync_copy(gidx.at[pl.ds(irow, irh)], gv)
            pltpu.sync_copy(sidx.at[pl.ds(irow, irh)], sv)

            @pl.loop(0, irh // depth)
            def _(g):
                c0 = g * depth
                hs = [pltpu.async_copy(table.at[gv.at[c0 + k]], rows[k],
                                       gsems[k]) for k in range(depth)]
                ws = []
                for k in range(depth):
                    hs[k].wait()
                    ws.append(pltpu.async_copy(rows[k], acc.at[sv.at[c0 + k]],
                                               ssems[k], add=True))
                for w in ws:
                    w.wait()

        plsc.subcore_barrier()
        stripe(acc, out.at[pl.ds(cid * _NP, _NP)])

    return agg


def _tc(body, out_shapes):
    return pl.pallas_call(body, out_shape=out_shapes)


def _dinv_body(degs_ref, x_ref, w_ref, dinv_ref, t_ref):
    deg = degs_ref[: _N, 0:1] + degs_ref[_NP : _NP + _N, 0:1] + 1.0
    dinv = lax.rsqrt(deg)
    dinv_ref[...] = dinv
    h = jnp.dot(x_ref[...], w_ref[...], preferred_element_type=_F32)
    t_ref[: _N] = h * dinv
    t_ref[_N:] = jnp.zeros((_NP - _N, _D), _F32)


def _bn_silu(y, g, b):
    mu = jnp.mean(y, axis=0, keepdims=True)
    var = jnp.mean((y - mu) ** 2, axis=0, keepdims=True)
    z = (y - mu) * lax.rsqrt(var + 1e-5) * g + b
    return z * jax.nn.sigmoid(z)


def _block1_body(acc_ref, t_ref, dinv_ref, b_ref, g_ref, be_ref, w_ref,
                 h1_ref, t1_ref):
    dinv = dinv_ref[...]
    gcn = (dinv * (acc_ref[: _N] + acc_ref[_NP : _NP + _N] + t_ref[: _N])
           + b_ref[...])
    h1 = _bn_silu(gcn, g_ref[...], be_ref[...])
    h1_ref[...] = h1
    t1_ref[: _N] = jnp.dot(h1, w_ref[...], preferred_element_type=_F32) * dinv
    t1_ref[_N:] = jnp.zeros((_NP - _N, _D), _F32)


def _block2_body(acc_ref, t_ref, dinv_ref, h1_ref, b_ref, g_ref, be_ref,
                 wp_ref, h2_ref, tp_ref):
    dinv = dinv_ref[...]
    gcn = (dinv * (acc_ref[: _N] + acc_ref[_NP : _NP + _N] + t_ref[: _N])
           + b_ref[...])
    h2 = h1_ref[...] + _bn_silu(gcn, g_ref[...], be_ref[...])
    h2_ref[...] = h2
    tp_ref[: _N] = jnp.dot(h2, wp_ref[...], preferred_element_type=_F32) * dinv
    tp_ref[_N:] = jnp.zeros((_NP - _N, _K1), _F32)


def _softmax(rows):
    m = jnp.max(rows, axis=-1, keepdims=True)
    e = jnp.exp(rows - m)
    return e / jnp.sum(e, axis=-1, keepdims=True)


def _pool_body(acc_ref, tp_ref, dinv_ref, bp_ref, h2_ref, s_ref, xn_ref):
    dinv = dinv_ref[...]
    logits = (dinv * (acc_ref[: _N] + acc_ref[_NP : _NP + _N]
                      + tp_ref[: _N]) + bp_ref[...])
    s = _softmax(logits)
    s_ref[: _N] = s
    s_ref[_N:] = jnp.zeros((_NP - _N, _K1), _F32)
    xn_ref[...] = lax.dot_general(s, h2_ref[...], (((0,), (0,)), ((), ())),
                                  preferred_element_type=_F32)


def _head_body(acc_ref, s_ref, xn_ref,
               w2_ref, b2_ref, g2_ref, be2_ref,
               w3_ref, b3_ref, g3_ref, be3_ref,
               wp2_ref, bp2_ref, wl_ref, bl_ref, logp_ref):
    a_s = acc_ref[: _N] + acc_ref[_NP : _NP + _N]
    a_new = lax.dot_general(s_ref[: _N], a_s, (((0,), (0,)), ((), ())),
                            preferred_element_type=_F32)
    ones = jnp.ones((_K1, 1), _F32)
    colsum = lax.dot_general(a_new, ones, (((0,), (0,)), ((), ())),
                             preferred_element_type=_F32)
    dinv2 = lax.rsqrt(colsum + 1.0)

    def conv(m, bias):
        agg = lax.dot_general(a_new, dinv2 * m, (((0,), (0,)), ((), ())),
                              preferred_element_type=_F32)
        return dinv2 * (agg + dinv2 * m) + bias

    h = xn_ref[...]
    y = conv(jnp.dot(h, w2_ref[...], preferred_element_type=_F32), b2_ref[...])
    h = _bn_silu(y, g2_ref[...], be2_ref[...])
    y = conv(jnp.dot(h, w3_ref[...], preferred_element_type=_F32), b3_ref[...])
    h = h + _bn_silu(y, g3_ref[...], be3_ref[...])
    lg = conv(jnp.dot(h, wp2_ref[...], preferred_element_type=_F32),
              bp2_ref[...])
    s2 = _softmax(lg)
    x2 = lax.dot_general(s2, h, (((0,), (0,)), ((), ())),
                         preferred_element_type=_F32)
    pooled = jnp.mean(x2, axis=0, keepdims=True)
    z = jnp.dot(pooled, wl_ref[...], preferred_element_type=_F32) + bl_ref[...]
    m = jnp.max(z, axis=-1, keepdims=True)
    lse = m + jnp.log(jnp.sum(jnp.exp(z - m), axis=-1, keepdims=True))
    logp_ref[...] = z - lse


def kernel(x, edge_index, batch, batch_ptr, params):
    p = params
    row = lambda v: v.reshape(1, -1)
    sd = lambda *s: jax.ShapeDtypeStruct(s, _F32)

    zeros_d = jnp.zeros((_RPS, _D), _F32)
    zeros_k = jnp.zeros((_RPS, _K1), _F32)
    zeros_s = jnp.zeros((_RPS, 16), _F32)
    ones_t = jnp.ones((_NP, 16), _F32)

    npad = _ERP * _C - _E
    pad = _N + (jnp.arange(npad, dtype=jnp.int32) % _NPAD)
    src = jnp.concatenate([edge_index[0], pad]).reshape(_ERP, _C)
    dst = jnp.concatenate([edge_index[1], pad]).reshape(_ERP, _C)

    degs = _agg_call(16, False)(ones_t, dst, dst, zeros_s)
    dinv, t0 = _tc(_dinv_body, [sd(_N, 1), sd(_NP, _D)])(degs, x, p['W0'])

    acc0 = _agg_call(_D)(t0, src, dst, zeros_d)
    h1, t1 = _tc(_block1_body, [sd(_N, _D), sd(_NP, _D)])(
        acc0, t0, dinv, row(p['b0']), row(p['g0']), row(p['be0']), p['W1'])

    acc1 = _agg_call(_D)(t1, src, dst, zeros_d)
    h2, tp = _tc(_block2_body, [sd(_N, _D), sd(_NP, _K1)])(
        acc1, t1, dinv, h1, row(p['b1']), row(p['g1']), row(p['be1']),
        p['Wp1'])

    accp = _agg_call(_K1, False)(tp, src, dst, zeros_k)
    s, xn = _tc(_pool_body, [sd(_NP, _K1), sd(_K1, _D)])(
        accp, tp, dinv, row(p['bp1']), h2)

    acc_as = _agg_call(_K1, False)(s, dst, src, zeros_k)
    logp = _tc(_head_body, sd(1, 10))(
        acc_as, s, xn,
        p['W2'], row(p['b2']), row(p['g2']), row(p['be2']),
        p['W3'], row(p['b3']), row(p['g3']), row(p['be3']),
        p['Wp2'], row(p['bp2']), p['Wl'], row(p['bl']))

    return logp, jnp.zeros((), _F32)
